# baseline (speedup 1.0000x reference)
import functools
import jax
import jax.numpy as jnp
from jax import lax
from jax.experimental import pallas as pl
from jax.experimental.pallas import tpu as pltpu
from jax.experimental.pallas import tpu_sc as plsc

N = 10000
D = 128
E = 160000
L1 = 64
L2 = 32
BLK = 1000

NSUB = 16
NPAD = 10112
RPT = NPAD // NSUB
CHUNK = 128
CPT = 80
EPT = CPT * CHUNK
EPAD = NSUB * EPT


def _l2n(x):
    nrm = jnp.sqrt(jnp.sum(x * x, axis=-1, keepdims=True))
    return x / jnp.maximum(nrm, 1e-12)


def _base_body(sp_ref, sn_ref, x_ref, cp_ref, cn_ref, wp_ref, bp_ref,
               wn_ref, bn_ref, h_ref):
    x = x_ref[...]
    cp = jnp.maximum(cp_ref[...], 1.0)
    cn = jnp.maximum(cn_ref[...], 1.0)
    aggp = sp_ref[...] / cp
    aggn = sn_ref[...] / cn
    wp = wp_ref[...]
    wn = wn_ref[...]
    up = (jnp.dot(aggp, wp[:D], preferred_element_type=jnp.float32)
          + jnp.dot(x, wp[D:], preferred_element_type=jnp.float32) + bp_ref[...])
    un = (jnp.dot(aggn, wn[:D], preferred_element_type=jnp.float32)
          + jnp.dot(x, wn[D:], preferred_element_type=jnp.float32) + bn_ref[...])
    hp = jnp.tanh(_l2n(up))
    hn = jnp.tanh(_l2n(un))
    h_ref[...] = jnp.concatenate([hp, hn], axis=1)


def _base_layer(sum_p, sum_n, X, cnt_p, cnt_n, Wp, bp, Wn, bn):
    grid = (N // BLK,)
    row = pl.BlockSpec((BLK, D), lambda i: (i, 0))
    col1 = pl.BlockSpec((BLK, 1), lambda i: (i, 0))
    full = lambda s: pl.BlockSpec(s, lambda i: (0, 0))
    return pl.pallas_call(
        _base_body,
        grid=grid,
        in_specs=[row, row, row, col1, col1,
                  full((2 * D, L1)), full((1, L1)), full((2 * D, L1)), full((1, L1))],
        out_specs=pl.BlockSpec((BLK, 2 * L1), lambda i: (i, 0)),
        out_shape=jax.ShapeDtypeStruct((N, 2 * L1), jnp.float32),
    )(sum_p, sum_n, X, cnt_p, cnt_n, Wp, bp.reshape(1, L1), Wn, bn.reshape(1, L1))


def _deep_body(s2p_ref, s2n_ref, h_ref, cp_ref, cn_ref, wpd_ref, bpd_ref,
               wnd_ref, bnd_ref, wr_ref, z_ref, zt_ref, p8_ref):
    h = h_ref[...]
    hp = h[:, :L1]
    hn = h[:, L1:]
    cp1 = cp_ref[...] + 1.0
    cn1 = cn_ref[...] + 1.0
    s2p = s2p_ref[...]
    s2n = s2n_ref[...]
    o1p = (s2p[:, :L1] + hp) / cp1
    o2p = (s2n[:, L1:] + hn) / cn1
    o1n = (s2p[:, L1:] + hn) / cp1
    o2n = (s2n[:, :L1] + hp) / cn1
    wpd = wpd_ref[...]
    wnd = wnd_ref[...]
    up = (jnp.dot(o1p, wpd[:L1], preferred_element_type=jnp.float32)
          + jnp.dot(o2p, wpd[L1:2 * L1], preferred_element_type=jnp.float32)
          + jnp.dot(hp, wpd[2 * L1:], preferred_element_type=jnp.float32)
          + bpd_ref[...])
    un = (jnp.dot(o1n, wnd[:L1], preferred_element_type=jnp.float32)
          + jnp.dot(o2n, wnd[L1:2 * L1], preferred_element_type=jnp.float32)
          + jnp.dot(hn, wnd[2 * L1:], preferred_element_type=jnp.float32)
          + bnd_ref[...])
    hp2 = jnp.tanh(_l2n(up))
    hn2 = jnp.tanh(_l2n(un))
    z = jnp.concatenate([hp2, hn2], axis=1)
    wr = wr_ref[...]
    p1 = jnp.dot(z, wr[:2 * L2], preferred_element_type=jnp.float32)
    p2 = jnp.dot(z, wr[2 * L2:], preferred_element_type=jnp.float32)
    n2 = jnp.sum(z * z, axis=1, keepdims=True)
    z_ref[...] = z
    zt_ref[...] = jnp.concatenate(
        [z, p1, p2, n2, jnp.zeros((z.shape[0], 57), jnp.float32)], axis=1)
    p8_ref[...] = jnp.concatenate(
        [p1, p2, n2, jnp.zeros((z.shape[0], 1), jnp.float32)], axis=1)


def _deep_layer(s2_p, s2_n, H, cnt_p, cnt_n, Wpd, bpd, Wnd, bnd, Wr):
    grid = (N // BLK,)
    row = pl.BlockSpec((BLK, D), lambda i: (i, 0))
    col1 = pl.BlockSpec((BLK, 1), lambda i: (i, 0))
    full = lambda s: pl.BlockSpec(s, lambda i: (0, 0))
    return pl.pallas_call(
        _deep_body,
        grid=grid,
        in_specs=[row, row, row, col1, col1,
                  full((3 * L1, L2)), full((1, L2)), full((3 * L1, L2)), full((1, L2)),
                  full((4 * L2, 3))],
        out_specs=[pl.BlockSpec((BLK, 2 * L2), lambda i: (i, 0)),
                   pl.BlockSpec((BLK, 128), lambda i: (i, 0)),
                   pl.BlockSpec((BLK, 8), lambda i: (i, 0))],
        out_shape=[jax.ShapeDtypeStruct((N, 2 * L2), jnp.float32),
                   jax.ShapeDtypeStruct((N, 128), jnp.float32),
                   jax.ShapeDtypeStruct((N, 8), jnp.float32)],
    )(s2_p, s2_n, H, cnt_p, cnt_n, Wpd, bpd.reshape(1, L2), Wnd, bnd.reshape(1, L2), Wr)


def _make_seg_sum(with_counts):
    mesh = plsc.VectorSubcoreMesh(core_axis_name="c", subcore_axis_name="s")
    out_type = [jax.ShapeDtypeStruct((2, NPAD, D), jnp.float32)]
    scratch = [
        pltpu.VMEM_SHARED((NPAD, D), jnp.float32),
        pltpu.VMEM((CHUNK, D), jnp.float32),
        pltpu.VMEM((CHUNK,), jnp.int32),
        pltpu.VMEM((CHUNK,), jnp.int32),
        pltpu.SemaphoreType.DMA,
    ]
    if with_counts:
        out_type.append(jax.ShapeDtypeStruct((2 * NPAD,), jnp.float32))
        scratch.append(pltpu.VMEM_SHARED((NPAD,), jnp.float32))
        scratch.append(pltpu.VMEM((CHUNK,), jnp.float32))

    @functools.partial(pl.kernel, out_type=out_type, mesh=mesh,
                       scratch_types=scratch)
    def seg(table_hbm, rows_hbm, cols_hbm, *rest):
        if with_counts:
            out_sum, out_cnt, acc, rows_v, ridx, cidx, sem, cntacc, ones_v = rest
        else:
            out_sum, acc, rows_v, ridx, cidx, sem = rest
        c = lax.axis_index("c")
        s = lax.axis_index("s")
        zero16 = jnp.zeros((16,), jnp.float32)

        def zrow(r, _):
            for q in range(D // 16):
                rows_v[r, pl.ds(q * 16, 16)] = zero16
            return 0
        lax.fori_loop(0, CHUNK, zrow, 0)

        base_r = s * RPT
        off = 0
        while off < RPT:
            sz = min(CHUNK, RPT - off)
            pltpu.sync_copy(rows_v.at[pl.ds(0, sz)],
                            acc.at[pl.ds(base_r + off, sz)])
            off += sz
        if with_counts:
            for q in range(CHUNK // 16):
                ones_v[pl.ds(q * 16, 16)] = zero16
            off = 0
            while off < RPT:
                sz = min(CHUNK, RPT - off)
                pltpu.sync_copy(ones_v.at[pl.ds(0, sz)],
                                cntacc.at[pl.ds(base_r + off, sz)])
                off += sz
            for q in range(CHUNK // 16):
                ones_v[pl.ds(q * 16, 16)] = jnp.ones((16,), jnp.float32)

        plsc.subcore_barrier()

        def chunk_body(t, _):
            base_e = s * EPT + t * CHUNK
            pltpu.sync_copy(rows_hbm.at[c, pl.ds(base_e, CHUNK)], ridx)
            pltpu.sync_copy(cols_hbm.at[c, pl.ds(base_e, CHUNK)], cidx)
            pltpu.async_copy(table_hbm.at[cidx], rows_v, sem).wait()
            pltpu.sync_copy(rows_v, acc.at[ridx], add=True)
            if with_counts:
                pltpu.sync_copy(ones_v, cntacc.at[ridx], add=True)
            return 0
        lax.fori_loop(0, CPT, chunk_body, 0)

        plsc.subcore_barrier()

        off = 0
        while off < RPT:
            sz = min(CHUNK, RPT - off)
            pltpu.sync_copy(acc.at[pl.ds(base_r + off, sz)],
                            rows_v.at[pl.ds(0, sz)])
            pltpu.sync_copy(rows_v.at[pl.ds(0, sz)],
                            out_sum.at[c, pl.ds(base_r + off, sz)])
            if with_counts:
                pltpu.sync_copy(cntacc.at[pl.ds(base_r + off, sz)],
                                ones_v.at[pl.ds(0, sz)])
                pltpu.sync_copy(ones_v.at[pl.ds(0, sz)],
                                out_cnt.at[pl.ds(c * NPAD + base_r + off, sz)])
            off += sz

    return seg


_seg_sum_cnt_sc = _make_seg_sum(True)
_seg_sum_sc = _make_seg_sum(False)


_LN2 = 0.6931471805599453


def _vlog(v):
    bits = lax.bitcast_convert_type(v, jnp.int32)
    ex = ((bits >> 23) & 255) - 127
    m = lax.bitcast_convert_type((bits & 0x7FFFFF) | 0x3F800000, jnp.float32)
    y = (m - 1.0) / (m + 1.0)
    y2 = y * y
    srs = y * (2.0 + y2 * (0.6666666666 + y2 * (0.4 + y2 * 0.2857142857)))
    return ex.astype(jnp.float32) * _LN2 + srs


def _perm(v, idx):
    return v.at[idx].get(mode="promise_in_bounds")


def _transpose16(rows, ii):
    lvl = list(rows)
    for stp in (8, 4, 2, 1):
        m = (ii & stp) == 0
        out = [None] * 16
        for i in range(16):
            if i & stp == 0:
                a, b = lvl[i], lvl[i ^ stp]
                out[i] = jnp.where(m, a, _perm(b, ii ^ stp))
                out[i ^ stp] = jnp.where(m, _perm(a, ii ^ stp), b)
        lvl = out
    return lvl


def _tree_reduce16(leaves, ii):
    lvl = list(leaves)
    for stp in (8, 4, 2, 1):
        m = (ii & stp) == 0
        lvl = [jnp.where(m, a + _perm(a, ii ^ stp), b + _perm(b, ii ^ stp))
               for a, b in zip(lvl[0::2], lvl[1::2])]
    sigma = (((ii & 1) << 3) | ((ii & 2) << 1) | ((ii & 4) >> 1)
             | ((ii & 8) >> 3))
    return _perm(lvl[0], sigma)


def _make_edge_loss():
    mesh = plsc.VectorSubcoreMesh(core_axis_name="c", subcore_axis_name="s")
    out_type = [jax.ShapeDtypeStruct((2 * NSUB * 16,), jnp.float32),
                jax.ShapeDtypeStruct((2 * NSUB * 16,), jnp.float32)]
    scratch = dict(
        Ri0=pltpu.VMEM((CHUNK, D), jnp.float32),
        Rj0=pltpu.VMEM((CHUNK, D), jnp.float32),
        Rk0=pltpu.VMEM((CHUNK, D), jnp.float32),
        Ri1=pltpu.VMEM((CHUNK, D), jnp.float32),
        Rj1=pltpu.VMEM((CHUNK, D), jnp.float32),
        Rk1=pltpu.VMEM((CHUNK, D), jnp.float32),
        ebuf0=pltpu.VMEM((3 * CHUNK,), jnp.int32),
        ebuf1=pltpu.VMEM((3 * CHUNK,), jnp.int32),
        tbuf0=pltpu.VMEM((3 * CHUNK,), jnp.int32),
        tbuf1=pltpu.VMEM((3 * CHUNK,), jnp.int32),
        hbuf=pltpu.VMEM((16,), jnp.float32),
        rbuf=pltpu.VMEM((16,), jnp.float32),
        sem=pltpu.SemaphoreType.DMA,
    )

    @functools.partial(pl.kernel, out_type=out_type, mesh=mesh,
                       scratch_types=list(scratch.values()))
    def edge_loss(zt_hbm, eidx_hbm, tgt_hbm, out_hinge, out_reg, *scr):
        sc = dict(zip(scratch.keys(), scr))
        rows = [(sc["Ri0"], sc["Rj0"], sc["Rk0"]),
                (sc["Ri1"], sc["Rj1"], sc["Rk1"])]
        ebufs = [sc["ebuf0"], sc["ebuf1"]]
        tbufs = [sc["tbuf0"], sc["tbuf1"]]
        sem = sc["sem"]
        c = lax.axis_index("c")
        s = lax.axis_index("s")
        lane = lax.iota(jnp.int32, 16) + (c - c)
        sign_v = jnp.full((16,), 1 - 2 * c, jnp.int32).astype(jnp.float32)

        def issue(tc, ph):
            off = ((c * NSUB + s) * CPT + tc) * (3 * CHUNK)
            pltpu.sync_copy(eidx_hbm.at[pl.ds(off, 3 * CHUNK)], ebufs[ph])
            pltpu.sync_copy(tgt_hbm.at[pl.ds(off, 3 * CHUNK)], tbufs[ph])
            return [pltpu.async_copy(
                zt_hbm.at[ebufs[ph].at[pl.ds(q * CHUNK, CHUNK)]],
                rows[ph][q], sem) for q in range(3)]

        def reg_term(pa, pb, tv):
            q0, q1, q2 = pa[0] + pb[0], pa[1] + pb[1], pa[2] + pb[2]
            m = jnp.maximum(jnp.maximum(q0, q1), q2)
            e = jnp.exp(q0 - m) + jnp.exp(q1 - m) + jnp.exp(q2 - m)
            pt = jnp.where(tv == 0, q0, jnp.where(tv == 1, q1, q2))
            return m + _vlog(e) - pt

        def compute(tc, ph, carry):
            hacc, racc = carry
            Ri, Rj, Rk = rows[ph]
            tbuf = tbufs[ph]
            base_e = s * EPT + tc * CHUNK

            def group_body(g, carry2):
                ha, ra = carry2
                gb = g * 16
                leaves, vi, vj, vk = [], [], [], []
                for l in range(16):
                    e = gb + l
                    acc = jnp.zeros((16,), jnp.float32)
                    for q in range(4):
                        zi = Ri[e, pl.ds(q * 16, 16)]
                        zj = Rj[e, pl.ds(q * 16, 16)]
                        zk = Rk[e, pl.ds(q * 16, 16)]
                        acc = acc + zi * (zk - zj)
                    leaves.append(acc)
                    vi.append(Ri[e, pl.ds(L1, 16)])
                    vj.append(Rj[e, pl.ds(L1, 16)])
                    vk.append(Rk[e, pl.ds(L1, 16)])
                dv = _tree_reduce16(leaves, lane)
                Ti = _transpose16(vi, lane)
                Tj = _transpose16(vj, lane)
                Tk = _transpose16(vk, lane)
                valid = (jnp.full((16,), base_e, jnp.int32) + gb + lane) < E
                hm = jnp.where(valid, sign_v, 0.0)
                hv = jnp.maximum(hm * ((Tj[6] - Tk[6]) + 2.0 * dv), 0.0)
                rm = jnp.where(valid, 1.0, 0.0)
                t0 = tbuf[pl.ds(gb, 16)]
                t1 = tbuf[pl.ds(CHUNK + gb, 16)]
                t2 = tbuf[pl.ds(2 * CHUNK + gb, 16)]
                r = (reg_term(Ti[0:3], Tj[3:6], t0)
                     + reg_term(Ti[0:3], Tk[3:6], t1)
                     + reg_term(Tj[0:3], Tk[3:6], t2))
                return (ha + hv, ra + rm * r)

            return lax.fori_loop(0, CHUNK // 16, group_body, (hacc, racc))

        def chunk_pair(t2i, carry):
            tc0 = 2 * t2i
            cps0 = issue(tc0, 0)
            cps1 = issue(tc0 + 1, 1)
            for cp in cps0:
                cp.wait()
            carry = compute(tc0, 0, carry)
            for cp in cps1:
                cp.wait()
            return compute(tc0 + 1, 1, carry)

        zf = jnp.zeros((16,), jnp.float32)
        hacc, racc = lax.fori_loop(0, CPT // 2, chunk_pair, (zf, zf))
        sc["hbuf"][pl.ds(0, 16)] = hacc
        sc["rbuf"][pl.ds(0, 16)] = racc
        out_off = (c * NSUB + s) * 16
        pltpu.sync_copy(sc["hbuf"], out_hinge.at[pl.ds(out_off, 16)])
        pltpu.sync_copy(sc["rbuf"], out_reg.at[pl.ds(out_off, 16)])

    return edge_loss


_edge_loss_sc = _make_edge_loss()


def kernel(X, positive_edges, negative_edges, target,
           W_pos_base, b_pos_base, W_neg_base, b_neg_base,
           W_pos_deep, b_pos_deep, W_neg_deep, b_neg_deep, regression_weights):
    skey = jax.random.key(7)
    pos_surr = jax.random.randint(jax.random.fold_in(skey, 0), (E,), 0, N, dtype=jnp.int32)
    neg_surr = jax.random.randint(jax.random.fold_in(skey, 1), (E,), 0, N, dtype=jnp.int32)

    rp, cp = positive_edges[0], positive_edges[1]
    rn, cn = negative_edges[0], negative_edges[1]

    rows2 = jnp.concatenate(
        [jnp.stack([rp, rn]), jnp.full((2, EPAD - E), N, jnp.int32)], axis=1)
    cols2 = jnp.concatenate(
        [jnp.stack([cp, cn]), jnp.zeros((2, EPAD - E), jnp.int32)], axis=1)

    sums, cnts = _seg_sum_cnt_sc(X, rows2, cols2)
    sum_p, sum_n = sums[0, :N], sums[1, :N]
    cnt_p, cnt_n = cnts[:N, None], cnts[NPAD:NPAD + N, None]

    H = _base_layer(sum_p, sum_n, X, cnt_p, cnt_n,
                    W_pos_base, b_pos_base, W_neg_base, b_neg_base)

    (sums2,) = _seg_sum_sc(H, rows2, cols2)
    s2_p, s2_n = sums2[0, :N], sums2[1, :N]

    z, ZT, P8 = _deep_layer(s2_p, s2_n, H, cnt_p, cnt_n,
                            W_pos_deep, b_pos_deep, W_neg_deep, b_neg_deep,
                            regression_weights)

    zpad = jnp.zeros((2, EPAD - E), jnp.int32)
    i2 = jnp.concatenate([jnp.stack([rp, rn]), zpad], axis=1)
    j2 = jnp.concatenate([jnp.stack([cp, cn]), zpad], axis=1)
    k2 = jnp.concatenate([jnp.stack([pos_surr, neg_surr]), zpad], axis=1)
    T6 = jnp.concatenate(
        [jnp.stack([target[0:E], target[4 * E:5 * E], target[5 * E:6 * E],
                    target[E:2 * E], target[2 * E:3 * E], target[3 * E:4 * E]]),
         jnp.zeros((6, EPAD - E), jnp.int32)], axis=1)

    eidx = jnp.stack([i2, j2, k2], axis=1)
    eidx = eidx.reshape(2, 3, NSUB, CPT, CHUNK).transpose(0, 2, 3, 1, 4)
    eidx = jnp.concatenate([eidx.reshape(-1),
                            jnp.zeros((3 * CHUNK,), jnp.int32)])
    tgt = T6.reshape(2, 3, NSUB, CPT, CHUNK).transpose(0, 2, 3, 1, 4)
    tgt = jnp.concatenate([tgt.reshape(-1),
                           jnp.zeros((3 * CHUNK,), jnp.int32)])
    hinge_out, reg_out = _edge_loss_sc(ZT, eidx, tgt)
    loss = (jnp.sum(reg_out) / (6.0 * E)
            + jnp.sum(hinge_out[:NSUB * 16]) / E
            + jnp.sum(hinge_out[NSUB * 16:]) / E)
    return loss, z


# speedup vs baseline: 1.3911x; 1.3911x over previous
import functools
import jax
import jax.numpy as jnp
from jax import lax
from jax.experimental import pallas as pl
from jax.experimental.pallas import tpu as pltpu
from jax.experimental.pallas import tpu_sc as plsc

N = 10000
D = 128
E = 160000
L1 = 64
L2 = 32
BLK = 1000

NSUB = 16
NPAD = 10112
RPT = NPAD // NSUB
CHUNK = 128
CPT = 79
EPT = CPT * CHUNK
EPAD = NSUB * EPT


def _l2n(x):
    nrm = jnp.sqrt(jnp.sum(x * x, axis=-1, keepdims=True))
    return x / jnp.maximum(nrm, 1e-12)


def _base_body(sp_ref, sn_ref, x_ref, cp_ref, cn_ref, wp_ref, bp_ref,
               wn_ref, bn_ref, h_ref):
    x = x_ref[...]
    cp = jnp.maximum(cp_ref[...], 1.0)
    cn = jnp.maximum(cn_ref[...], 1.0)
    aggp = sp_ref[...] / cp
    aggn = sn_ref[...] / cn
    wp = wp_ref[...]
    wn = wn_ref[...]
    up = (jnp.dot(aggp, wp[:D], preferred_element_type=jnp.float32)
          + jnp.dot(x, wp[D:], preferred_element_type=jnp.float32) + bp_ref[...])
    un = (jnp.dot(aggn, wn[:D], preferred_element_type=jnp.float32)
          + jnp.dot(x, wn[D:], preferred_element_type=jnp.float32) + bn_ref[...])
    hp = jnp.tanh(_l2n(up))
    hn = jnp.tanh(_l2n(un))
    h_ref[...] = jnp.concatenate([hp, hn], axis=1)


def _base_layer(sum_p, sum_n, X, cnt_p, cnt_n, Wp, bp, Wn, bn):
    grid = (N // BLK,)
    row = pl.BlockSpec((BLK, D), lambda i: (i, 0))
    col1 = pl.BlockSpec((BLK, 1), lambda i: (i, 0))
    full = lambda s: pl.BlockSpec(s, lambda i: (0, 0))
    return pl.pallas_call(
        _base_body,
        grid=grid,
        in_specs=[row, row, row, col1, col1,
                  full((2 * D, L1)), full((1, L1)), full((2 * D, L1)), full((1, L1))],
        out_specs=pl.BlockSpec((BLK, 2 * L1), lambda i: (i, 0)),
        out_shape=jax.ShapeDtypeStruct((N, 2 * L1), jnp.float32),
    )(sum_p, sum_n, X, cnt_p, cnt_n, Wp, bp.reshape(1, L1), Wn, bn.reshape(1, L1))


def _deep_body(s2p_ref, s2n_ref, h_ref, cp_ref, cn_ref, wpd_ref, bpd_ref,
               wnd_ref, bnd_ref, wr_ref, z_ref, zt_ref, p8_ref):
    h = h_ref[...]
    hp = h[:, :L1]
    hn = h[:, L1:]
    cp1 = cp_ref[...] + 1.0
    cn1 = cn_ref[...] + 1.0
    s2p = s2p_ref[...]
    s2n = s2n_ref[...]
    o1p = (s2p[:, :L1] + hp) / cp1
    o2p = (s2n[:, L1:] + hn) / cn1
    o1n = (s2p[:, L1:] + hn) / cp1
    o2n = (s2n[:, :L1] + hp) / cn1
    wpd = wpd_ref[...]
    wnd = wnd_ref[...]
    up = (jnp.dot(o1p, wpd[:L1], preferred_element_type=jnp.float32)
          + jnp.dot(o2p, wpd[L1:2 * L1], preferred_element_type=jnp.float32)
          + jnp.dot(hp, wpd[2 * L1:], preferred_element_type=jnp.float32)
          + bpd_ref[...])
    un = (jnp.dot(o1n, wnd[:L1], preferred_element_type=jnp.float32)
          + jnp.dot(o2n, wnd[L1:2 * L1], preferred_element_type=jnp.float32)
          + jnp.dot(hn, wnd[2 * L1:], preferred_element_type=jnp.float32)
          + bnd_ref[...])
    hp2 = jnp.tanh(_l2n(up))
    hn2 = jnp.tanh(_l2n(un))
    z = jnp.concatenate([hp2, hn2], axis=1)
    wr = wr_ref[...]
    p1 = jnp.dot(z, wr[:2 * L2], preferred_element_type=jnp.float32)
    p2 = jnp.dot(z, wr[2 * L2:], preferred_element_type=jnp.float32)
    n2 = jnp.sum(z * z, axis=1, keepdims=True)
    z_ref[...] = z
    zt_ref[...] = jnp.concatenate(
        [z, p1, p2, n2, jnp.zeros((z.shape[0], 57), jnp.float32)], axis=1)
    p8_ref[...] = jnp.concatenate(
        [p1, p2, n2, jnp.zeros((z.shape[0], 1), jnp.float32)], axis=1)


def _deep_layer(s2_p, s2_n, H, cnt_p, cnt_n, Wpd, bpd, Wnd, bnd, Wr):
    grid = (N // BLK,)
    row = pl.BlockSpec((BLK, D), lambda i: (i, 0))
    col1 = pl.BlockSpec((BLK, 1), lambda i: (i, 0))
    full = lambda s: pl.BlockSpec(s, lambda i: (0, 0))
    return pl.pallas_call(
        _deep_body,
        grid=grid,
        in_specs=[row, row, row, col1, col1,
                  full((3 * L1, L2)), full((1, L2)), full((3 * L1, L2)), full((1, L2)),
                  full((4 * L2, 3))],
        out_specs=[pl.BlockSpec((BLK, 2 * L2), lambda i: (i, 0)),
                   pl.BlockSpec((BLK, 128), lambda i: (i, 0)),
                   pl.BlockSpec((BLK, 8), lambda i: (i, 0))],
        out_shape=[jax.ShapeDtypeStruct((N, 2 * L2), jnp.float32),
                   jax.ShapeDtypeStruct((N, 128), jnp.float32),
                   jax.ShapeDtypeStruct((N, 8), jnp.float32)],
    )(s2_p, s2_n, H, cnt_p, cnt_n, Wpd, bpd.reshape(1, L2), Wnd, bnd.reshape(1, L2), Wr)


def _make_seg_sum(with_counts):
    mesh = plsc.VectorSubcoreMesh(core_axis_name="c", subcore_axis_name="s")
    out_type = [jax.ShapeDtypeStruct((2, NPAD, D), jnp.float32)]
    scratch = [
        pltpu.VMEM_SHARED((NPAD, D), jnp.float32),
        pltpu.VMEM((CHUNK, D), jnp.float32),
        pltpu.VMEM((CHUNK,), jnp.int32),
        pltpu.VMEM((CHUNK,), jnp.int32),
        pltpu.SemaphoreType.DMA,
    ]
    if with_counts:
        out_type.append(jax.ShapeDtypeStruct((2 * NPAD,), jnp.float32))
        scratch.append(pltpu.VMEM_SHARED((NPAD,), jnp.float32))
        scratch.append(pltpu.VMEM((CHUNK,), jnp.float32))

    @functools.partial(pl.kernel, out_type=out_type, mesh=mesh,
                       scratch_types=scratch)
    def seg(table_hbm, rows_hbm, cols_hbm, *rest):
        if with_counts:
            out_sum, out_cnt, acc, rows_v, ridx, cidx, sem, cntacc, ones_v = rest
        else:
            out_sum, acc, rows_v, ridx, cidx, sem = rest
        c = lax.axis_index("c")
        s = lax.axis_index("s")
        zero16 = jnp.zeros((16,), jnp.float32)

        def zrow(r, _):
            for q in range(D // 16):
                rows_v[r, pl.ds(q * 16, 16)] = zero16
            return 0
        lax.fori_loop(0, CHUNK, zrow, 0)

        base_r = s * RPT
        off = 0
        while off < RPT:
            sz = min(CHUNK, RPT - off)
            pltpu.sync_copy(rows_v.at[pl.ds(0, sz)],
                            acc.at[pl.ds(base_r + off, sz)])
            off += sz
        if with_counts:
            for q in range(CHUNK // 16):
                ones_v[pl.ds(q * 16, 16)] = zero16
            off = 0
            while off < RPT:
                sz = min(CHUNK, RPT - off)
                pltpu.sync_copy(ones_v.at[pl.ds(0, sz)],
                                cntacc.at[pl.ds(base_r + off, sz)])
                off += sz
            for q in range(CHUNK // 16):
                ones_v[pl.ds(q * 16, 16)] = jnp.ones((16,), jnp.float32)

        plsc.subcore_barrier()

        def chunk_body(t, _):
            base_e = s * EPT + t * CHUNK
            pltpu.sync_copy(rows_hbm.at[c, pl.ds(base_e, CHUNK)], ridx)
            pltpu.sync_copy(cols_hbm.at[c, pl.ds(base_e, CHUNK)], cidx)
            pltpu.async_copy(table_hbm.at[cidx], rows_v, sem).wait()
            pltpu.sync_copy(rows_v, acc.at[ridx], add=True)
            if with_counts:
                pltpu.sync_copy(ones_v, cntacc.at[ridx], add=True)
            return 0
        lax.fori_loop(0, CPT, chunk_body, 0)

        plsc.subcore_barrier()

        off = 0
        while off < RPT:
            sz = min(CHUNK, RPT - off)
            pltpu.sync_copy(acc.at[pl.ds(base_r + off, sz)],
                            rows_v.at[pl.ds(0, sz)])
            pltpu.sync_copy(rows_v.at[pl.ds(0, sz)],
                            out_sum.at[c, pl.ds(base_r + off, sz)])
            if with_counts:
                pltpu.sync_copy(cntacc.at[pl.ds(base_r + off, sz)],
                                ones_v.at[pl.ds(0, sz)])
                pltpu.sync_copy(ones_v.at[pl.ds(0, sz)],
                                out_cnt.at[pl.ds(c * NPAD + base_r + off, sz)])
            off += sz

    return seg


_seg_sum_cnt_sc = _make_seg_sum(True)
_seg_sum_sc = _make_seg_sum(False)


_LN2 = 0.6931471805599453


def _vlog(v):
    bits = lax.bitcast_convert_type(v, jnp.int32)
    ex = ((bits >> 23) & 255) - 127
    m = lax.bitcast_convert_type((bits & 0x7FFFFF) | 0x3F800000, jnp.float32)
    y = (m - 1.0) / (m + 1.0)
    y2 = y * y
    srs = y * (2.0 + y2 * (0.6666666666 + y2 * (0.4 + y2 * 0.2857142857)))
    return ex.astype(jnp.float32) * _LN2 + srs


def _perm(v, idx):
    return v.at[idx].get(mode="promise_in_bounds")


def _transpose16(rows, ii):
    lvl = list(rows)
    for stp in (8, 4, 2, 1):
        m = (ii & stp) == 0
        out = [None] * 16
        for i in range(16):
            if i & stp == 0:
                a, b = lvl[i], lvl[i ^ stp]
                out[i] = jnp.where(m, a, _perm(b, ii ^ stp))
                out[i ^ stp] = jnp.where(m, _perm(a, ii ^ stp), b)
        lvl = out
    return lvl


def _tree_reduce16(leaves, ii):
    lvl = list(leaves)
    for stp in (8, 4, 2, 1):
        m = (ii & stp) == 0
        lvl = [jnp.where(m, a + _perm(a, ii ^ stp), b + _perm(b, ii ^ stp))
               for a, b in zip(lvl[0::2], lvl[1::2])]
    sigma = (((ii & 1) << 3) | ((ii & 2) << 1) | ((ii & 4) >> 1)
             | ((ii & 8) >> 3))
    return _perm(lvl[0], sigma)


def _make_edge_loss():
    mesh = plsc.VectorSubcoreMesh(core_axis_name="c", subcore_axis_name="s")
    out_type = [jax.ShapeDtypeStruct((2 * NSUB * 16,), jnp.float32),
                jax.ShapeDtypeStruct((2 * NSUB * 16,), jnp.float32)]
    scratch = dict(
        Ri=pltpu.VMEM((CHUNK, D), jnp.float32),
        Rj=pltpu.VMEM((CHUNK, D), jnp.float32),
        Rk=pltpu.VMEM((CHUNK, D), jnp.float32),
        ebuf=pltpu.VMEM((3 * CHUNK,), jnp.int32),
        tbuf=pltpu.VMEM((3 * CHUNK,), jnp.int32),
        hbuf=pltpu.VMEM((16,), jnp.float32),
        rbuf=pltpu.VMEM((16,), jnp.float32),
        sem=pltpu.SemaphoreType.DMA,
    )

    @functools.partial(pl.kernel, out_type=out_type, mesh=mesh,
                       scratch_types=list(scratch.values()))
    def edge_loss(zt_hbm, eidx_hbm, tgt_hbm, out_hinge, out_reg, *scr):
        sc = dict(zip(scratch.keys(), scr))
        Ri, Rj, Rk = sc["Ri"], sc["Rj"], sc["Rk"]
        ebuf, tbuf = sc["ebuf"], sc["tbuf"]
        sem = sc["sem"]
        c = lax.axis_index("c")
        s = lax.axis_index("s")
        lane = lax.iota(jnp.int32, 16) + (c - c)
        sign_v = jnp.full((16,), 1 - 2 * c, jnp.int32).astype(jnp.float32)

        def chunk_body(t, carry):
            hacc, racc = carry
            base_e = s * EPT + t * CHUNK
            off = ((c * NSUB + s) * CPT + t) * (3 * CHUNK)
            pltpu.sync_copy(eidx_hbm.at[pl.ds(off, 3 * CHUNK)], ebuf)
            pltpu.sync_copy(tgt_hbm.at[pl.ds(off, 3 * CHUNK)], tbuf)
            cps = [
                pltpu.async_copy(
                    zt_hbm.at[ebuf.at[pl.ds(0, CHUNK)]], Ri, sem),
                pltpu.async_copy(
                    zt_hbm.at[ebuf.at[pl.ds(CHUNK, CHUNK)]], Rj, sem),
                pltpu.async_copy(
                    zt_hbm.at[ebuf.at[pl.ds(2 * CHUNK, CHUNK)]], Rk, sem),
            ]
            for cp in cps:
                cp.wait()

            def reg_term(pa, pb, tv):
                q0, q1, q2 = pa[0] + pb[0], pa[1] + pb[1], pa[2] + pb[2]
                m = jnp.maximum(jnp.maximum(q0, q1), q2)
                e = jnp.exp(q0 - m) + jnp.exp(q1 - m) + jnp.exp(q2 - m)
                pt = jnp.where(tv == 0, q0, jnp.where(tv == 1, q1, q2))
                return m + _vlog(e) - pt

            def group_body(g, carry2):
                ha, ra = carry2
                gb = g * 16
                leaves, vi, vj, vk = [], [], [], []
                for l in range(16):
                    e = gb + l
                    acc = jnp.zeros((16,), jnp.float32)
                    for q in range(4):
                        zi = Ri[e, pl.ds(q * 16, 16)]
                        zj = Rj[e, pl.ds(q * 16, 16)]
                        zk = Rk[e, pl.ds(q * 16, 16)]
                        acc = acc + zi * (zk - zj)
                    leaves.append(acc)
                    vi.append(Ri[e, pl.ds(L1, 16)])
                    vj.append(Rj[e, pl.ds(L1, 16)])
                    vk.append(Rk[e, pl.ds(L1, 16)])
                dv = _tree_reduce16(leaves, lane)
                Ti = _transpose16(vi, lane)
                Tj = _transpose16(vj, lane)
                Tk = _transpose16(vk, lane)
                valid = (jnp.full((16,), base_e, jnp.int32) + gb + lane) < E
                hm = jnp.where(valid, sign_v, 0.0)
                hv = jnp.maximum(hm * ((Tj[6] - Tk[6]) + 2.0 * dv), 0.0)
                rm = jnp.where(valid, 1.0, 0.0)
                t0 = tbuf[pl.ds(gb, 16)]
                t1 = tbuf[pl.ds(CHUNK + gb, 16)]
                t2 = tbuf[pl.ds(2 * CHUNK + gb, 16)]
                r = (reg_term(Ti[0:3], Tj[3:6], t0)
                     + reg_term(Ti[0:3], Tk[3:6], t1)
                     + reg_term(Tj[0:3], Tk[3:6], t2))
                return (ha + hv, ra + rm * r)

            return lax.fori_loop(0, CHUNK // 16, group_body, (hacc, racc))

        zf = jnp.zeros((16,), jnp.float32)
        hacc, racc = lax.fori_loop(0, CPT, chunk_body, (zf, zf))
        sc["hbuf"][pl.ds(0, 16)] = hacc
        sc["rbuf"][pl.ds(0, 16)] = racc
        out_off = (c * NSUB + s) * 16
        pltpu.sync_copy(sc["hbuf"], out_hinge.at[pl.ds(out_off, 16)])
        pltpu.sync_copy(sc["rbuf"], out_reg.at[pl.ds(out_off, 16)])

    return edge_loss


_edge_loss_sc = _make_edge_loss()


def kernel(X, positive_edges, negative_edges, target,
           W_pos_base, b_pos_base, W_neg_base, b_neg_base,
           W_pos_deep, b_pos_deep, W_neg_deep, b_neg_deep, regression_weights):
    skey = jax.random.key(7)
    pos_surr = jax.random.randint(jax.random.fold_in(skey, 0), (E,), 0, N, dtype=jnp.int32)
    neg_surr = jax.random.randint(jax.random.fold_in(skey, 1), (E,), 0, N, dtype=jnp.int32)

    rp, cp = positive_edges[0], positive_edges[1]
    rn, cn = negative_edges[0], negative_edges[1]

    rows2 = jnp.concatenate(
        [jnp.stack([rp, rn]), jnp.full((2, EPAD - E), N, jnp.int32)], axis=1)
    cols2 = jnp.concatenate(
        [jnp.stack([cp, cn]), jnp.zeros((2, EPAD - E), jnp.int32)], axis=1)

    sums, cnts = _seg_sum_cnt_sc(X, rows2, cols2)
    sum_p, sum_n = sums[0, :N], sums[1, :N]
    cnt_p, cnt_n = cnts[:N, None], cnts[NPAD:NPAD + N, None]

    H = _base_layer(sum_p, sum_n, X, cnt_p, cnt_n,
                    W_pos_base, b_pos_base, W_neg_base, b_neg_base)

    (sums2,) = _seg_sum_sc(H, rows2, cols2)
    s2_p, s2_n = sums2[0, :N], sums2[1, :N]

    z, ZT, P8 = _deep_layer(s2_p, s2_n, H, cnt_p, cnt_n,
                            W_pos_deep, b_pos_deep, W_neg_deep, b_neg_deep,
                            regression_weights)

    zpad = jnp.zeros((2, EPAD - E), jnp.int32)
    i2 = jnp.concatenate([jnp.stack([rp, rn]), zpad], axis=1)
    j2 = jnp.concatenate([jnp.stack([cp, cn]), zpad], axis=1)
    k2 = jnp.concatenate([jnp.stack([pos_surr, neg_surr]), zpad], axis=1)
    T6 = jnp.concatenate(
        [jnp.stack([target[0:E], target[4 * E:5 * E], target[5 * E:6 * E],
                    target[E:2 * E], target[2 * E:3 * E], target[3 * E:4 * E]]),
         jnp.zeros((6, EPAD - E), jnp.int32)], axis=1)

    eidx = jnp.stack([i2, j2, k2], axis=1)
    eidx = eidx.reshape(2, 3, NSUB, CPT, CHUNK).transpose(0, 2, 3, 1, 4)
    eidx = eidx.reshape(-1)
    tgt = T6.reshape(2, 3, NSUB, CPT, CHUNK).transpose(0, 2, 3, 1, 4)
    tgt = tgt.reshape(-1)
    hinge_out, reg_out = _edge_loss_sc(ZT, eidx, tgt)
    loss = (jnp.sum(reg_out) / (6.0 * E)
            + jnp.sum(hinge_out[:NSUB * 16]) / E
            + jnp.sum(hinge_out[NSUB * 16:]) / E)
    return loss, z
